# baseline (speedup 1.0000x reference)
import jax
import jax.numpy as jnp
from jax import lax
from jax.experimental import pallas as pl
from jax.experimental.pallas import tpu as pltpu
from jax.experimental.pallas import tpu_sc as plsc

_NC = 2
_NS = 16
_NW = _NC * _NS
_LB = 128
_UNTILED = pltpu.CompilerParams(use_tc_tiling_on_sc=False)
_UNTILED_NL = pltpu.CompilerParams(
    use_tc_tiling_on_sc=False, needs_layout_passes=False
)


def _deg_route_kernel_fn(n_nodes, n_batches, rows_sh):
    rows_per_sub = rows_sh // _NS
    zcopies = rows_per_sub // _LB
    half = n_nodes // 2
    cap = n_batches * _LB

    def body(src3, dst3, degpart, lAs, lAd, lBs, lBd, cA, cB,
             src_v, dst_v, as1, ad1, bs1, bd1, ad2, bd2, ones_v, zbuf, cnt_v,
             deg_sh):
        cid = lax.axis_index("c")
        sid = lax.axis_index("s")
        wid = sid * _NC + cid

        zero16 = jnp.zeros((16,), jnp.float32)
        one16 = jnp.ones((16,), jnp.float32)

        def fill(i, _):
            zbuf[i, 0:16] = zero16
            ones_v[i, 0:16] = one16
            return 0

        lax.fori_loop(0, _LB, fill, 0)

        for k in range(zcopies):
            pltpu.sync_copy(zbuf, deg_sh.at[pl.ds(sid * rows_per_sub + k * _LB, _LB)])
        plsc.subcore_barrier()

        pltpu.sync_copy(src3.at[pl.ds(wid * n_batches, n_batches)], src_v)
        pltpu.sync_copy(dst3.at[pl.ds(wid * n_batches, n_batches)], dst_v)

        def step(j, _):
            pltpu.sync_copy(ones_v, deg_sh.at[dst_v.at[j]], add=True)
            return 0

        lax.fori_loop(0, n_batches, step, 0)

        z16i = jnp.zeros((16,), jnp.int32)
        t16i = jnp.full((16,), half, jnp.int32)

        def tfill(i, _):
            as1[pl.ds(i * 16, 16)] = z16i
            bs1[pl.ds(i * 16, 16)] = z16i
            ad1[pl.ds(i * 16, 16)] = t16i
            bd1[pl.ds(i * 16, 16)] = t16i
            return 0

        lax.fori_loop(0, cap // 16, tfill, 0)

        def route_row(r, carry):
            offA, offB = carry
            for c in range(_LB // 16):
                s = src_v[r, pl.ds(c * 16, 16)]
                d = dst_v[r, pl.ds(c * 16, 16)]
                m = d < half
                plsc.store_compressed(as1.at[pl.ds(offA, 16)], s, mask=m)
                plsc.store_compressed(ad1.at[pl.ds(offA, 16)], d, mask=m)
                mb = jnp.logical_not(m)
                plsc.store_compressed(bs1.at[pl.ds(offB, 16)], s, mask=mb)
                plsc.store_compressed(bd1.at[pl.ds(offB, 16)], d - half, mask=mb)
                offA = offA + jnp.max(plsc.all_reduce_population_count(m))
                offB = offB + jnp.max(plsc.all_reduce_population_count(mb))
            return offA, offB

        offA, offB = lax.fori_loop(
            0, n_batches, route_row,
            (jnp.zeros((), jnp.int32), jnp.zeros((), jnp.int32)),
        )

        def to2d(r, _):
            for c in range(_LB // 16):
                ad2[r, pl.ds(c * 16, 16)] = ad1[pl.ds(r * _LB + c * 16, 16)]
                bd2[r, pl.ds(c * 16, 16)] = bd1[pl.ds(r * _LB + c * 16, 16)]
            return 0

        lax.fori_loop(0, n_batches, to2d, 0)

        plsc.subcore_barrier()

        pltpu.sync_copy(
            deg_sh.at[pl.ds(sid * rows_per_sub, rows_per_sub)],
            degpart.at[pl.ds(cid * rows_sh + sid * rows_per_sub, rows_per_sub)],
        )
        pltpu.sync_copy(as1, lAs.at[wid])
        pltpu.sync_copy(bs1, lBs.at[wid])
        pltpu.sync_copy(ad2, lAd.at[pl.ds(wid * n_batches, n_batches)])
        pltpu.sync_copy(bd2, lBd.at[pl.ds(wid * n_batches, n_batches)])

        for i in range(_LB // 16):
            cnt_v[pl.ds(i * 16, 16)] = jnp.full((16,), offA, jnp.int32)
        pltpu.sync_copy(cnt_v, cA.at[wid])
        for i in range(_LB // 16):
            cnt_v[pl.ds(i * 16, 16)] = jnp.full((16,), offB, jnp.int32)
        pltpu.sync_copy(cnt_v, cB.at[wid])

    return pl.kernel(
        body,
        out_type=(
            jax.ShapeDtypeStruct((_NC * rows_sh, 16), jnp.float32),
            jax.ShapeDtypeStruct((_NW, cap), jnp.int32),
            jax.ShapeDtypeStruct((_NW * n_batches, _LB), jnp.int32),
            jax.ShapeDtypeStruct((_NW, cap), jnp.int32),
            jax.ShapeDtypeStruct((_NW * n_batches, _LB), jnp.int32),
            jax.ShapeDtypeStruct((_NW, _LB), jnp.int32),
            jax.ShapeDtypeStruct((_NW, _LB), jnp.int32),
        ),
        mesh=plsc.VectorSubcoreMesh(core_axis_name="c", subcore_axis_name="s"),
        scratch_types=[
            pltpu.VMEM((n_batches, _LB), jnp.int32),
            pltpu.VMEM((n_batches, _LB), jnp.int32),
            pltpu.VMEM((cap,), jnp.int32),
            pltpu.VMEM((cap,), jnp.int32),
            pltpu.VMEM((cap,), jnp.int32),
            pltpu.VMEM((cap,), jnp.int32),
            pltpu.VMEM((n_batches, _LB), jnp.int32),
            pltpu.VMEM((n_batches, _LB), jnp.int32),
            pltpu.VMEM((_LB, 16), jnp.float32),
            pltpu.VMEM((_LB, 16), jnp.float32),
            pltpu.VMEM((_LB,), jnp.int32),
            pltpu.VMEM_SHARED((rows_sh, 16), jnp.float32),
        ],
        compiler_params=_UNTILED_NL,
    )


def _agg_kernel_fn(n_nodes, d, n_batches, rows_half):
    stripe = rows_half // _NS
    zrows = 160
    zcopies = stripe // zrows
    nbuf = 4
    n_per_sub = n_nodes // _NS
    half = n_nodes // 2

    def body(h128, lAs, lAd, lBs, lBd, cA, cB, ag,
             srcq_v, dstq_v, rows_v, zbuf, cnt_v, sg0, sg1, sg2, sg3,
             agg_sh, h_sh):
        cid = lax.axis_index("c")
        sid = lax.axis_index("s")
        sgs = (sg0, sg1, sg2, sg3)

        zero16 = jnp.zeros((16,), jnp.float32)

        def fill(i, _):
            for j in range(d // 16):
                zbuf[i, pl.ds(j * 16, 16)] = zero16
            return 0

        lax.fori_loop(0, zrows, fill, 0)

        pltpu.sync_copy(
            h128.at[pl.ds(sid * n_per_sub, n_per_sub), pl.ds(0, d)],
            h_sh.at[pl.ds(sid * n_per_sub, n_per_sub)],
        )
        for k in range(zcopies):
            pltpu.sync_copy(zbuf, agg_sh.at[pl.ds(sid * stripe + k * zrows, zrows)])
        plsc.subcore_barrier()

        def run_lists(Ls, Ld, Cc):
            for q in range(2):
                w = 2 * sid + q
                pltpu.sync_copy(Ls.at[w], srcq_v)
                pltpu.sync_copy(Ld.at[pl.ds(w * n_batches, n_batches)], dstq_v)
                pltpu.sync_copy(Cc.at[w], cnt_v)
                cnt = jnp.max(cnt_v[0:16])
                nbq = (cnt + _LB - 1) // _LB

                for b in range(nbuf):
                    @pl.when(b < nbq)
                    def _():
                        pltpu.async_copy(
                            h_sh.at[srcq_v.at[pl.ds(b * _LB, _LB)]],
                            rows_v.at[b], sgs[b],
                        )

                def group(jj, _):
                    for b in range(nbuf):
                        j = jj * nbuf + b

                        @pl.when(j < nbq)
                        def _():
                            pltpu.make_async_copy(
                                h_sh.at[srcq_v.at[pl.ds(j * _LB, _LB)]],
                                rows_v.at[b], sgs[b],
                            ).wait()
                            pltpu.sync_copy(
                                rows_v.at[b], agg_sh.at[dstq_v.at[j]], add=True
                            )

                        @pl.when(j + nbuf < nbq)
                        def _():
                            pltpu.async_copy(
                                h_sh.at[srcq_v.at[pl.ds((j + nbuf) * _LB, _LB)]],
                                rows_v.at[b], sgs[b],
                            )

                    return 0

                lax.fori_loop(0, n_batches // nbuf, group, 0)

        @pl.when(cid == 0)
        def _():
            run_lists(lAs, lAd, cA)

        @pl.when(cid == 1)
        def _():
            run_lists(lBs, lBd, cB)

        plsc.subcore_barrier()
        pltpu.sync_copy(
            agg_sh.at[pl.ds(sid * stripe, stripe)],
            ag.at[pl.ds(cid * rows_half + sid * stripe, stripe)],
        )

    cap = n_batches * _LB
    return pl.kernel(
        body,
        out_type=jax.ShapeDtypeStruct((_NC * rows_half, d), jnp.float32),
        mesh=plsc.VectorSubcoreMesh(core_axis_name="c", subcore_axis_name="s"),
        scratch_types=[
            pltpu.VMEM((cap,), jnp.int32),
            pltpu.VMEM((n_batches, _LB), jnp.int32),
            pltpu.VMEM((nbuf, _LB, d), jnp.float32),
            pltpu.VMEM((zrows, d), jnp.float32),
            pltpu.VMEM((_LB,), jnp.int32),
            pltpu.SemaphoreType.DMA,
            pltpu.SemaphoreType.DMA,
            pltpu.SemaphoreType.DMA,
            pltpu.SemaphoreType.DMA,
            pltpu.VMEM_SHARED((rows_half, d), jnp.float32),
            pltpu.VMEM_SHARED((n_nodes, d), jnp.float32),
        ],
        compiler_params=_UNTILED_NL,
    )


def _dis_col(dp_ref, n):
    deg = 1.0 + dp_ref[0][0:n, 0:1] + dp_ref[1][0:n, 0:1]
    return lax.rsqrt(deg)


def _tc1_body(x_ref, w1_ref, dp_ref, h1_ref):
    n, d = h1_ref.shape[0], w1_ref.shape[1]
    dis = _dis_col(dp_ref, n)
    h = jnp.dot(x_ref[...], w1_ref[...], preferred_element_type=jnp.float32)
    h1_ref[:, 0:d] = h * dis
    h1_ref[:, d:] = jnp.zeros((n, h1_ref.shape[1] - d), jnp.float32)


def _agg_total(ag_ref, h_ref, n, d):
    half = n // 2
    rows_half = ag_ref.shape[0] // 2
    return (
        jnp.concatenate(
            [ag_ref[0:half], ag_ref[rows_half:rows_half + half]], axis=0
        )
        + h_ref[0:n, 0:d]
    )


def _tc2_body(ag_ref, h1_ref, dp_ref, w2_ref, b1_ref, h2_ref):
    n, d = h2_ref.shape[0], w2_ref.shape[1]
    dis = _dis_col(dp_ref, n)
    out1 = _agg_total(ag_ref, h1_ref, n, d) * dis + b1_ref[...]
    h2_ref[:, 0:d] = jnp.dot(out1, w2_ref[...], preferred_element_type=jnp.float32) * dis
    h2_ref[:, d:] = jnp.zeros((n, h2_ref.shape[1] - d), jnp.float32)


def _tc3_body(ag_ref, h2_ref, dp_ref, b2_ref, out_ref):
    n, d = out_ref.shape
    dis = _dis_col(dp_ref, n)
    out_ref[...] = _agg_total(ag_ref, h2_ref, n, d) * dis + b2_ref[...]


def kernel(x, edge_index, W1, b1, W2, b2):
    n, d_in = x.shape
    d_out = W1.shape[1]
    e = edge_index.shape[1]

    n_batches = 8 * (-(-e // (_NW * _LB * 8)))
    e_pad = _NW * _LB * n_batches
    rows_sh = _NS * _LB * (-(-(n + 1) // (_NS * _LB)))
    rows_half = _NS * 8 * (-(-(n // 2 + 1) // (_NS * 8)))
    assert d_out % 16 == 0 and n % 2 == 0 and n % _NS == 0

    src = jnp.concatenate([edge_index[0], jnp.zeros((e_pad - e,), jnp.int32)]).reshape(-1, _LB)
    dst = jnp.concatenate([edge_index[1], jnp.full((e_pad - e,), n, jnp.int32)]).reshape(-1, _LB)

    dp_flat, lAs, lAd, lBs, lBd, cA, cB = _deg_route_kernel_fn(
        n, n_batches, rows_sh
    )(src, dst)
    dp = dp_flat.reshape(_NC, rows_sh, 16)

    agg_raw = _agg_kernel_fn(n, d_out, n_batches, rows_half)

    def agg(h128):
        return agg_raw(h128, lAs, lAd, lBs, lBd, cA, cB)

    b1r = b1.reshape(1, d_out)
    b2r = b2.reshape(1, d_out)

    h1 = pl.pallas_call(
        _tc1_body,
        out_shape=jax.ShapeDtypeStruct((n, 128), jnp.float32),
    )(x, W1, dp)

    ag1 = agg(h1)

    h2 = pl.pallas_call(
        _tc2_body,
        out_shape=jax.ShapeDtypeStruct((n, 128), jnp.float32),
    )(ag1, h1, dp, W2, b1r)

    ag2 = agg(h2)

    out = pl.pallas_call(
        _tc3_body,
        out_shape=jax.ShapeDtypeStruct((n, d_out), jnp.float32),
    )(ag2, h2, dp, b2r)

    return out


# speedup vs baseline: 1.1139x; 1.1139x over previous
import jax
import jax.numpy as jnp
from jax import lax
from jax.experimental import pallas as pl
from jax.experimental.pallas import tpu as pltpu
from jax.experimental.pallas import tpu_sc as plsc

_NC = 2
_NS = 16
_NW = _NC * _NS
_LB = 128
_UNTILED = pltpu.CompilerParams(use_tc_tiling_on_sc=False)
_UNTILED_NL = pltpu.CompilerParams(
    use_tc_tiling_on_sc=False, needs_layout_passes=False
)


def _deg_route_kernel_fn(n_nodes, n_batches, rows_sh):
    rows_per_sub = rows_sh // _NS
    zcopies = rows_per_sub // _LB
    half = n_nodes // 2
    cap = n_batches * _LB

    def body(ei2, degpart, lAs, lAd, lBs, lBd, cA, cB,
             src_v, dst_v, as1, ad1, bs1, bd1, ad2, bd2, ones_v, zbuf, cnt_v,
             deg_sh):
        eb = ei2.shape[0] // 2
        cid = lax.axis_index("c")
        sid = lax.axis_index("s")
        wid = sid * _NC + cid
        nb_w = jnp.clip(eb - wid * n_batches, 0, n_batches)

        zero16 = jnp.zeros((16,), jnp.float32)
        one16 = jnp.ones((16,), jnp.float32)

        def fill(i, _):
            zbuf[i, 0:16] = zero16
            ones_v[i, 0:16] = one16
            return 0

        lax.fori_loop(0, _LB, fill, 0)

        for k in range(zcopies):
            pltpu.sync_copy(zbuf, deg_sh.at[pl.ds(sid * rows_per_sub + k * _LB, _LB)])
        plsc.subcore_barrier()

        full = eb % n_batches == 0
        last_rows = eb % n_batches if not full else n_batches

        @pl.when(jnp.logical_or(wid * n_batches + n_batches <= eb, full))
        def _():
            pltpu.sync_copy(ei2.at[pl.ds(wid * n_batches, n_batches)], src_v)
            pltpu.sync_copy(ei2.at[pl.ds(eb + wid * n_batches, n_batches)], dst_v)

        if not full:
            @pl.when(wid * n_batches + n_batches > eb)
            def _():
                pltpu.sync_copy(
                    ei2.at[pl.ds((_NW - 1) * n_batches, last_rows)],
                    src_v.at[pl.ds(0, last_rows)],
                )
                pltpu.sync_copy(
                    ei2.at[pl.ds(eb + (_NW - 1) * n_batches, last_rows)],
                    dst_v.at[pl.ds(0, last_rows)],
                )

        def step(j, _):
            pltpu.sync_copy(ones_v, deg_sh.at[dst_v.at[j]], add=True)
            return 0

        lax.fori_loop(0, nb_w, step, 0)

        z16i = jnp.zeros((16,), jnp.int32)
        t16i = jnp.full((16,), half, jnp.int32)

        def tfill(i, _):
            as1[pl.ds(i * 16, 16)] = z16i
            bs1[pl.ds(i * 16, 16)] = z16i
            ad1[pl.ds(i * 16, 16)] = t16i
            bd1[pl.ds(i * 16, 16)] = t16i
            return 0

        lax.fori_loop(0, cap // 16, tfill, 0)

        def route_row(r, carry):
            offA, offB = carry
            for c in range(_LB // 16):
                s = src_v[r, pl.ds(c * 16, 16)]
                d = dst_v[r, pl.ds(c * 16, 16)]
                m = d < half
                plsc.store_compressed(as1.at[pl.ds(offA, 16)], s, mask=m)
                plsc.store_compressed(ad1.at[pl.ds(offA, 16)], d, mask=m)
                mb = jnp.logical_not(m)
                plsc.store_compressed(bs1.at[pl.ds(offB, 16)], s, mask=mb)
                plsc.store_compressed(bd1.at[pl.ds(offB, 16)], d - half, mask=mb)
                offA = offA + jnp.max(plsc.all_reduce_population_count(m))
                offB = offB + jnp.max(plsc.all_reduce_population_count(mb))
            return offA, offB

        offA, offB = lax.fori_loop(
            0, nb_w, route_row,
            (jnp.zeros((), jnp.int32), jnp.zeros((), jnp.int32)),
        )

        def to2d(r, _):
            for c in range(_LB // 16):
                ad2[r, pl.ds(c * 16, 16)] = ad1[pl.ds(r * _LB + c * 16, 16)]
                bd2[r, pl.ds(c * 16, 16)] = bd1[pl.ds(r * _LB + c * 16, 16)]
            return 0

        lax.fori_loop(0, n_batches, to2d, 0)

        plsc.subcore_barrier()

        pltpu.sync_copy(
            deg_sh.at[pl.ds(sid * rows_per_sub, rows_per_sub)],
            degpart.at[pl.ds(cid * rows_sh + sid * rows_per_sub, rows_per_sub)],
        )
        pltpu.sync_copy(as1, lAs.at[wid])
        pltpu.sync_copy(bs1, lBs.at[wid])
        pltpu.sync_copy(ad2, lAd.at[pl.ds(wid * n_batches, n_batches)])
        pltpu.sync_copy(bd2, lBd.at[pl.ds(wid * n_batches, n_batches)])

        for i in range(_LB // 16):
            cnt_v[pl.ds(i * 16, 16)] = jnp.full((16,), offA, jnp.int32)
        pltpu.sync_copy(cnt_v, cA.at[wid])
        for i in range(_LB // 16):
            cnt_v[pl.ds(i * 16, 16)] = jnp.full((16,), offB, jnp.int32)
        pltpu.sync_copy(cnt_v, cB.at[wid])

    return pl.kernel(
        body,
        out_type=(
            jax.ShapeDtypeStruct((_NC * rows_sh, 16), jnp.float32),
            jax.ShapeDtypeStruct((_NW, cap), jnp.int32),
            jax.ShapeDtypeStruct((_NW * n_batches, _LB), jnp.int32),
            jax.ShapeDtypeStruct((_NW, cap), jnp.int32),
            jax.ShapeDtypeStruct((_NW * n_batches, _LB), jnp.int32),
            jax.ShapeDtypeStruct((_NW, _LB), jnp.int32),
            jax.ShapeDtypeStruct((_NW, _LB), jnp.int32),
        ),
        mesh=plsc.VectorSubcoreMesh(core_axis_name="c", subcore_axis_name="s"),
        scratch_types=[
            pltpu.VMEM((n_batches, _LB), jnp.int32),
            pltpu.VMEM((n_batches, _LB), jnp.int32),
            pltpu.VMEM((cap,), jnp.int32),
            pltpu.VMEM((cap,), jnp.int32),
            pltpu.VMEM((cap,), jnp.int32),
            pltpu.VMEM((cap,), jnp.int32),
            pltpu.VMEM((n_batches, _LB), jnp.int32),
            pltpu.VMEM((n_batches, _LB), jnp.int32),
            pltpu.VMEM((_LB, 16), jnp.float32),
            pltpu.VMEM((_LB, 16), jnp.float32),
            pltpu.VMEM((_LB,), jnp.int32),
            pltpu.VMEM_SHARED((rows_sh, 16), jnp.float32),
        ],
        compiler_params=_UNTILED_NL,
    )


def _agg_kernel_fn(n_nodes, d, n_batches, rows_half):
    stripe = rows_half // _NS
    zrows = 160
    zcopies = stripe // zrows
    nbuf = 4
    n_per_sub = n_nodes // _NS
    half = n_nodes // 2

    def body(h128, lAs, lAd, lBs, lBd, cA, cB, ag,
             srcq_v, dstq_v, rows_v, zbuf, cnt_v, sg0, sg1, sg2, sg3,
             agg_sh, h_sh):
        cid = lax.axis_index("c")
        sid = lax.axis_index("s")
        sgs = (sg0, sg1, sg2, sg3)

        zero16 = jnp.zeros((16,), jnp.float32)

        def fill(i, _):
            for j in range(d // 16):
                zbuf[i, pl.ds(j * 16, 16)] = zero16
            return 0

        lax.fori_loop(0, zrows, fill, 0)

        pltpu.sync_copy(
            h128.at[pl.ds(sid * n_per_sub, n_per_sub), pl.ds(0, d)],
            h_sh.at[pl.ds(sid * n_per_sub, n_per_sub)],
        )
        for k in range(zcopies):
            pltpu.sync_copy(zbuf, agg_sh.at[pl.ds(sid * stripe + k * zrows, zrows)])
        plsc.subcore_barrier()

        def run_lists(Ls, Ld, Cc):
            for q in range(2):
                w = 2 * sid + q
                pltpu.sync_copy(Ls.at[w], srcq_v)
                pltpu.sync_copy(Ld.at[pl.ds(w * n_batches, n_batches)], dstq_v)
                pltpu.sync_copy(Cc.at[w], cnt_v)
                cnt = jnp.max(cnt_v[0:16])
                nbq = (cnt + _LB - 1) // _LB

                for b in range(nbuf):
                    @pl.when(b < nbq)
                    def _():
                        pltpu.async_copy(
                            h_sh.at[srcq_v.at[pl.ds(b * _LB, _LB)]],
                            rows_v.at[b], sgs[b],
                        )

                def group(jj, _):
                    for b in range(nbuf):
                        j = jj * nbuf + b

                        @pl.when(j < nbq)
                        def _():
                            pltpu.make_async_copy(
                                h_sh.at[srcq_v.at[pl.ds(j * _LB, _LB)]],
                                rows_v.at[b], sgs[b],
                            ).wait()
                            pltpu.sync_copy(
                                rows_v.at[b], agg_sh.at[dstq_v.at[j]], add=True
                            )

                        @pl.when(j + nbuf < nbq)
                        def _():
                            pltpu.async_copy(
                                h_sh.at[srcq_v.at[pl.ds((j + nbuf) * _LB, _LB)]],
                                rows_v.at[b], sgs[b],
                            )

                    return 0

                lax.fori_loop(0, n_batches // nbuf, group, 0)

        @pl.when(cid == 0)
        def _():
            run_lists(lAs, lAd, cA)

        @pl.when(cid == 1)
        def _():
            run_lists(lBs, lBd, cB)

        plsc.subcore_barrier()
        pltpu.sync_copy(
            agg_sh.at[pl.ds(sid * stripe, stripe)],
            ag.at[pl.ds(cid * rows_half + sid * stripe, stripe)],
        )

    cap = n_batches * _LB
    return pl.kernel(
        body,
        out_type=jax.ShapeDtypeStruct((_NC * rows_half, d), jnp.float32),
        mesh=plsc.VectorSubcoreMesh(core_axis_name="c", subcore_axis_name="s"),
        scratch_types=[
            pltpu.VMEM((cap,), jnp.int32),
            pltpu.VMEM((n_batches, _LB), jnp.int32),
            pltpu.VMEM((nbuf, _LB, d), jnp.float32),
            pltpu.VMEM((zrows, d), jnp.float32),
            pltpu.VMEM((_LB,), jnp.int32),
            pltpu.SemaphoreType.DMA,
            pltpu.SemaphoreType.DMA,
            pltpu.SemaphoreType.DMA,
            pltpu.SemaphoreType.DMA,
            pltpu.VMEM_SHARED((rows_half, d), jnp.float32),
            pltpu.VMEM_SHARED((n_nodes, d), jnp.float32),
        ],
        compiler_params=_UNTILED_NL,
    )


def _dis_col(dp_ref, n):
    deg = 1.0 + dp_ref[0][0:n, 0:1] + dp_ref[1][0:n, 0:1]
    return lax.rsqrt(deg)


def _tc1_body(x_ref, w1_ref, dp_ref, h1_ref):
    n, d = h1_ref.shape[0], w1_ref.shape[1]
    dis = _dis_col(dp_ref, n)
    h = jnp.dot(x_ref[...], w1_ref[...], preferred_element_type=jnp.float32)
    h1_ref[:, 0:d] = h * dis
    h1_ref[:, d:] = jnp.zeros((n, h1_ref.shape[1] - d), jnp.float32)


def _agg_total(ag_ref, h_ref, n, d):
    half = n // 2
    rows_half = ag_ref.shape[0] // 2
    return (
        jnp.concatenate(
            [ag_ref[0:half], ag_ref[rows_half:rows_half + half]], axis=0
        )
        + h_ref[0:n, 0:d]
    )


def _tc2_body(ag_ref, h1_ref, dp_ref, w2_ref, b1_ref, h2_ref):
    n, d = h2_ref.shape[0], w2_ref.shape[1]
    dis = _dis_col(dp_ref, n)
    out1 = _agg_total(ag_ref, h1_ref, n, d) * dis + b1_ref[...]
    h2_ref[:, 0:d] = jnp.dot(out1, w2_ref[...], preferred_element_type=jnp.float32) * dis
    h2_ref[:, d:] = jnp.zeros((n, h2_ref.shape[1] - d), jnp.float32)


def _tc3_body(ag_ref, h2_ref, dp_ref, b2_ref, out_ref):
    n, d = out_ref.shape
    dis = _dis_col(dp_ref, n)
    out_ref[...] = _agg_total(ag_ref, h2_ref, n, d) * dis + b2_ref[...]


def kernel(x, edge_index, W1, b1, W2, b2):
    n, d_in = x.shape
    d_out = W1.shape[1]
    e = edge_index.shape[1]

    n_batches = 8 * (-(-e // (_NW * _LB * 8)))
    rows_sh = _NS * _LB * (-(-(n + 1) // (_NS * _LB)))
    rows_half = _NS * 8 * (-(-(n // 2 + 1) // (_NS * 8)))
    eb = e // _LB
    assert d_out % 16 == 0 and n % 2 == 0 and n % _NS == 0
    assert e % _LB == 0 and (_NW - 1) * n_batches <= eb

    ei2 = edge_index.reshape(2 * eb, _LB)

    dp_flat, lAs, lAd, lBs, lBd, cA, cB = _deg_route_kernel_fn(
        n, n_batches, rows_sh
    )(ei2)
    dp = dp_flat.reshape(_NC, rows_sh, 16)

    agg_raw = _agg_kernel_fn(n, d_out, n_batches, rows_half)

    def agg(h128):
        return agg_raw(h128, lAs, lAd, lBs, lBd, cA, cB)

    b1r = b1.reshape(1, d_out)
    b2r = b2.reshape(1, d_out)

    h1 = pl.pallas_call(
        _tc1_body,
        out_shape=jax.ShapeDtypeStruct((n, 128), jnp.float32),
    )(x, W1, dp)

    ag1 = agg(h1)

    h2 = pl.pallas_call(
        _tc2_body,
        out_shape=jax.ShapeDtypeStruct((n, 128), jnp.float32),
    )(ag1, h1, dp, W2, b1r)

    ag2 = agg(h2)

    out = pl.pallas_call(
        _tc3_body,
        out_shape=jax.ShapeDtypeStruct((n, d_out), jnp.float32),
    )(ag2, h2, dp, b2r)

    return out


# speedup vs baseline: 1.1557x; 1.0375x over previous
import jax
import jax.numpy as jnp
from jax import lax
from jax.experimental import pallas as pl
from jax.experimental.pallas import tpu as pltpu
from jax.experimental.pallas import tpu_sc as plsc

_NC = 2
_NS = 16
_NW = _NC * _NS
_LB = 128
_UNTILED = pltpu.CompilerParams(use_tc_tiling_on_sc=False)
_UNTILED_NL = pltpu.CompilerParams(
    use_tc_tiling_on_sc=False, needs_layout_passes=False
)


def _deg_route_kernel_fn(n_nodes, n_batches, rows_sh):
    rows_per_sub = rows_sh // _NS
    zcopies = rows_per_sub // _LB
    half = n_nodes // 2
    cap = n_batches * _LB

    def body(ei2, degpart, lAs, lAd, lBs, lBd, cA, cB,
             src_v, dst_v, as1, ad1, bs1, bd1, ad2, bd2, ones_v, zbuf, cnt_v,
             sdeg, deg_sh):
        eb = ei2.shape[0] // 2
        cid = lax.axis_index("c")
        sid = lax.axis_index("s")
        wid = sid * _NC + cid
        nb_w = jnp.clip(eb - wid * n_batches, 0, n_batches)

        zero16 = jnp.zeros((16,), jnp.float32)
        one16 = jnp.ones((16,), jnp.float32)

        def fill(i, _):
            zbuf[i, 0:16] = zero16
            ones_v[i, 0:16] = one16
            return 0

        lax.fori_loop(0, _LB, fill, 0)

        for k in range(zcopies):
            pltpu.sync_copy(zbuf, deg_sh.at[pl.ds(sid * rows_per_sub + k * _LB, _LB)])
        plsc.subcore_barrier()

        full = eb % n_batches == 0
        last_rows = eb % n_batches if not full else n_batches

        @pl.when(jnp.logical_or(wid * n_batches + n_batches <= eb, full))
        def _():
            pltpu.sync_copy(ei2.at[pl.ds(wid * n_batches, n_batches)], src_v)
            pltpu.sync_copy(ei2.at[pl.ds(eb + wid * n_batches, n_batches)], dst_v)

        if not full:
            @pl.when(wid * n_batches + n_batches > eb)
            def _():
                pltpu.sync_copy(
                    ei2.at[pl.ds((_NW - 1) * n_batches, last_rows)],
                    src_v.at[pl.ds(0, last_rows)],
                )
                pltpu.sync_copy(
                    ei2.at[pl.ds(eb + (_NW - 1) * n_batches, last_rows)],
                    dst_v.at[pl.ds(0, last_rows)],
                )


        z16i = jnp.zeros((16,), jnp.int32)
        t16i = jnp.full((16,), half, jnp.int32)

        def tfill(i, _):
            as1[pl.ds(i * 16, 16)] = z16i
            bs1[pl.ds(i * 16, 16)] = z16i
            ad1[pl.ds(i * 16, 16)] = t16i
            bd1[pl.ds(i * 16, 16)] = t16i
            return 0

        lax.fori_loop(0, cap // 16, tfill, 0)

        def route_row(r, carry):
            offA, offB = carry
            pltpu.async_copy(ones_v, deg_sh.at[dst_v.at[r]], sdeg, add=True)
            for c in range(_LB // 16):
                s = src_v[r, pl.ds(c * 16, 16)]
                d = dst_v[r, pl.ds(c * 16, 16)]
                m = d < half
                plsc.store_compressed(as1.at[pl.ds(offA, 16)], s, mask=m)
                plsc.store_compressed(ad1.at[pl.ds(offA, 16)], d, mask=m)
                mb = jnp.logical_not(m)
                plsc.store_compressed(bs1.at[pl.ds(offB, 16)], s, mask=mb)
                plsc.store_compressed(bd1.at[pl.ds(offB, 16)], d - half, mask=mb)
                pop = jnp.max(plsc.all_reduce_population_count(m))
                offA = offA + pop
                offB = offB + (16 - pop)
            return offA, offB

        offA, offB = lax.fori_loop(
            0, nb_w, route_row,
            (jnp.zeros((), jnp.int32), jnp.zeros((), jnp.int32)),
        )

        def drain(j, _):
            pltpu.make_async_copy(ones_v, deg_sh.at[dst_v.at[0]], sdeg).wait()
            return 0

        lax.fori_loop(0, nb_w, drain, 0)

        def to2d(r, _):
            for c in range(_LB // 16):
                ad2[r, pl.ds(c * 16, 16)] = ad1[pl.ds(r * _LB + c * 16, 16)]
                bd2[r, pl.ds(c * 16, 16)] = bd1[pl.ds(r * _LB + c * 16, 16)]
            return 0

        lax.fori_loop(0, n_batches, to2d, 0)

        plsc.subcore_barrier()

        pltpu.sync_copy(
            deg_sh.at[pl.ds(sid * rows_per_sub, rows_per_sub)],
            degpart.at[pl.ds(cid * rows_sh + sid * rows_per_sub, rows_per_sub)],
        )
        pltpu.sync_copy(as1, lAs.at[wid])
        pltpu.sync_copy(bs1, lBs.at[wid])
        pltpu.sync_copy(ad2, lAd.at[pl.ds(wid * n_batches, n_batches)])
        pltpu.sync_copy(bd2, lBd.at[pl.ds(wid * n_batches, n_batches)])

        for i in range(_LB // 16):
            cnt_v[pl.ds(i * 16, 16)] = jnp.full((16,), offA, jnp.int32)
        pltpu.sync_copy(cnt_v, cA.at[wid])
        for i in range(_LB // 16):
            cnt_v[pl.ds(i * 16, 16)] = jnp.full((16,), offB, jnp.int32)
        pltpu.sync_copy(cnt_v, cB.at[wid])

    return pl.kernel(
        body,
        out_type=(
            jax.ShapeDtypeStruct((_NC * rows_sh, 16), jnp.float32),
            jax.ShapeDtypeStruct((_NW, cap), jnp.int32),
            jax.ShapeDtypeStruct((_NW * n_batches, _LB), jnp.int32),
            jax.ShapeDtypeStruct((_NW, cap), jnp.int32),
            jax.ShapeDtypeStruct((_NW * n_batches, _LB), jnp.int32),
            jax.ShapeDtypeStruct((_NW, _LB), jnp.int32),
            jax.ShapeDtypeStruct((_NW, _LB), jnp.int32),
        ),
        mesh=plsc.VectorSubcoreMesh(core_axis_name="c", subcore_axis_name="s"),
        scratch_types=[
            pltpu.VMEM((n_batches, _LB), jnp.int32),
            pltpu.VMEM((n_batches, _LB), jnp.int32),
            pltpu.VMEM((cap,), jnp.int32),
            pltpu.VMEM((cap,), jnp.int32),
            pltpu.VMEM((cap,), jnp.int32),
            pltpu.VMEM((cap,), jnp.int32),
            pltpu.VMEM((n_batches, _LB), jnp.int32),
            pltpu.VMEM((n_batches, _LB), jnp.int32),
            pltpu.VMEM((_LB, 16), jnp.float32),
            pltpu.VMEM((_LB, 16), jnp.float32),
            pltpu.VMEM((_LB,), jnp.int32),
            pltpu.SemaphoreType.DMA,
            pltpu.VMEM_SHARED((rows_sh, 16), jnp.float32),
        ],
        compiler_params=_UNTILED_NL,
    )


def _agg_kernel_fn(n_nodes, d, n_batches, rows_half):
    stripe = rows_half // _NS
    zrows = 160
    zcopies = stripe // zrows
    nbuf = 4
    n_per_sub = n_nodes // _NS
    half = n_nodes // 2

    def body(h128, lAs, lAd, lBs, lBd, cA, cB, ag,
             srcq_v, dstq_v, rows_v, zbuf, cnt_v, sg0, sg1, sg2, sg3,
             agg_sh, h_sh):
        cid = lax.axis_index("c")
        sid = lax.axis_index("s")
        sgs = (sg0, sg1, sg2, sg3)

        zero16 = jnp.zeros((16,), jnp.float32)

        def fill(i, _):
            for j in range(d // 16):
                zbuf[i, pl.ds(j * 16, 16)] = zero16
            return 0

        lax.fori_loop(0, zrows, fill, 0)

        pltpu.sync_copy(
            h128.at[pl.ds(sid * n_per_sub, n_per_sub), pl.ds(0, d)],
            h_sh.at[pl.ds(sid * n_per_sub, n_per_sub)],
        )
        for k in range(zcopies):
            pltpu.sync_copy(zbuf, agg_sh.at[pl.ds(sid * stripe + k * zrows, zrows)])
        plsc.subcore_barrier()

        def run_lists(Ls, Ld, Cc):
            for q in range(2):
                w = 2 * sid + q
                pltpu.sync_copy(Ls.at[w], srcq_v)
                pltpu.sync_copy(Ld.at[pl.ds(w * n_batches, n_batches)], dstq_v)
                pltpu.sync_copy(Cc.at[w], cnt_v)
                cnt = jnp.max(cnt_v[0:16])
                nbq = (cnt + _LB - 1) // _LB

                for b in range(nbuf):
                    @pl.when(b < nbq)
                    def _():
                        pltpu.async_copy(
                            h_sh.at[srcq_v.at[pl.ds(b * _LB, _LB)]],
                            rows_v.at[b], sgs[b],
                        )

                def group(jj, _):
                    for b in range(nbuf):
                        j = jj * nbuf + b

                        @pl.when(j < nbq)
                        def _():
                            pltpu.make_async_copy(
                                h_sh.at[srcq_v.at[pl.ds(j * _LB, _LB)]],
                                rows_v.at[b], sgs[b],
                            ).wait()
                            pltpu.sync_copy(
                                rows_v.at[b], agg_sh.at[dstq_v.at[j]], add=True
                            )

                        @pl.when(j + nbuf < nbq)
                        def _():
                            pltpu.async_copy(
                                h_sh.at[srcq_v.at[pl.ds((j + nbuf) * _LB, _LB)]],
                                rows_v.at[b], sgs[b],
                            )

                    return 0

                lax.fori_loop(0, n_batches // nbuf, group, 0)

        @pl.when(cid == 0)
        def _():
            run_lists(lAs, lAd, cA)

        @pl.when(cid == 1)
        def _():
            run_lists(lBs, lBd, cB)

        plsc.subcore_barrier()
        pltpu.sync_copy(
            agg_sh.at[pl.ds(sid * stripe, stripe)],
            ag.at[pl.ds(cid * rows_half + sid * stripe, stripe)],
        )

    cap = n_batches * _LB
    return pl.kernel(
        body,
        out_type=jax.ShapeDtypeStruct((_NC * rows_half, d), jnp.float32),
        mesh=plsc.VectorSubcoreMesh(core_axis_name="c", subcore_axis_name="s"),
        scratch_types=[
            pltpu.VMEM((cap,), jnp.int32),
            pltpu.VMEM((n_batches, _LB), jnp.int32),
            pltpu.VMEM((nbuf, _LB, d), jnp.float32),
            pltpu.VMEM((zrows, d), jnp.float32),
            pltpu.VMEM((_LB,), jnp.int32),
            pltpu.SemaphoreType.DMA,
            pltpu.SemaphoreType.DMA,
            pltpu.SemaphoreType.DMA,
            pltpu.SemaphoreType.DMA,
            pltpu.VMEM_SHARED((rows_half, d), jnp.float32),
            pltpu.VMEM_SHARED((n_nodes, d), jnp.float32),
        ],
        compiler_params=_UNTILED_NL,
    )


def _dis_col(dp_ref, n):
    deg = 1.0 + dp_ref[0][0:n, 0:1] + dp_ref[1][0:n, 0:1]
    return lax.rsqrt(deg)


def _tc1_body(x_ref, w1_ref, dp_ref, h1_ref):
    n, d = h1_ref.shape[0], w1_ref.shape[1]
    dis = _dis_col(dp_ref, n)
    h = jnp.dot(x_ref[...], w1_ref[...], preferred_element_type=jnp.float32)
    h1_ref[:, 0:d] = h * dis
    h1_ref[:, d:] = jnp.zeros((n, h1_ref.shape[1] - d), jnp.float32)


def _agg_total(ag_ref, h_ref, n, d):
    half = n // 2
    rows_half = ag_ref.shape[0] // 2
    return (
        jnp.concatenate(
            [ag_ref[0:half], ag_ref[rows_half:rows_half + half]], axis=0
        )
        + h_ref[0:n, 0:d]
    )


def _tc2_body(ag_ref, h1_ref, dp_ref, w2_ref, b1_ref, h2_ref):
    n, d = h2_ref.shape[0], w2_ref.shape[1]
    dis = _dis_col(dp_ref, n)
    out1 = _agg_total(ag_ref, h1_ref, n, d) * dis + b1_ref[...]
    h2_ref[:, 0:d] = jnp.dot(out1, w2_ref[...], preferred_element_type=jnp.float32) * dis
    h2_ref[:, d:] = jnp.zeros((n, h2_ref.shape[1] - d), jnp.float32)


def _tc3_body(ag_ref, h2_ref, dp_ref, b2_ref, out_ref):
    n, d = out_ref.shape
    dis = _dis_col(dp_ref, n)
    out_ref[...] = _agg_total(ag_ref, h2_ref, n, d) * dis + b2_ref[...]


def kernel(x, edge_index, W1, b1, W2, b2):
    n, d_in = x.shape
    d_out = W1.shape[1]
    e = edge_index.shape[1]

    n_batches = 8 * (-(-e // (_NW * _LB * 8)))
    rows_sh = _NS * _LB * (-(-(n + 1) // (_NS * _LB)))
    rows_half = _NS * 8 * (-(-(n // 2 + 1) // (_NS * 8)))
    eb = e // _LB
    assert d_out % 16 == 0 and n % 2 == 0 and n % _NS == 0
    assert e % _LB == 0 and (_NW - 1) * n_batches <= eb

    ei2 = edge_index.reshape(2 * eb, _LB)

    dp_flat, lAs, lAd, lBs, lBd, cA, cB = _deg_route_kernel_fn(
        n, n_batches, rows_sh
    )(ei2)
    dp = dp_flat.reshape(_NC, rows_sh, 16)

    agg_raw = _agg_kernel_fn(n, d_out, n_batches, rows_half)

    def agg(h128):
        return agg_raw(h128, lAs, lAd, lBs, lBd, cA, cB)

    b1r = b1.reshape(1, d_out)
    b2r = b2.reshape(1, d_out)

    h1 = pl.pallas_call(
        _tc1_body,
        out_shape=jax.ShapeDtypeStruct((n, 128), jnp.float32),
    )(x, W1, dp)

    ag1 = agg(h1)

    h2 = pl.pallas_call(
        _tc2_body,
        out_shape=jax.ShapeDtypeStruct((n, 128), jnp.float32),
    )(ag1, h1, dp, W2, b1r)

    ag2 = agg(h2)

    out = pl.pallas_call(
        _tc3_body,
        out_shape=jax.ShapeDtypeStruct((n, d_out), jnp.float32),
    )(ag2, h2, dp, b2r)

    return out


# speedup vs baseline: 1.1570x; 1.0011x over previous
import jax
import jax.numpy as jnp
from jax import lax
from jax.experimental import pallas as pl
from jax.experimental.pallas import tpu as pltpu
from jax.experimental.pallas import tpu_sc as plsc

_NC = 2
_NS = 16
_NW = _NC * _NS
_LB = 128
_UNTILED = pltpu.CompilerParams(use_tc_tiling_on_sc=False)
_UNTILED_NL = pltpu.CompilerParams(
    use_tc_tiling_on_sc=False, needs_layout_passes=False
)


def _deg_route_kernel_fn(n_nodes, n_batches, rows_sh):
    rows_per_sub = rows_sh // _NS
    zcopies = rows_per_sub // _LB
    half = n_nodes // 2
    cap = n_batches * _LB

    def body(ei2, degpart, lAs, lAd, lBs, lBd, cA, cB,
             src_v, dst_v, as1, ad1, bs1, bd1, ad2, bd2, ones_v, zbuf, cnt_v,
             sdeg, deg_sh):
        eb = ei2.shape[0] // 2
        cid = lax.axis_index("c")
        sid = lax.axis_index("s")
        wid = sid * _NC + cid
        nb_w = jnp.clip(eb - wid * n_batches, 0, n_batches)

        zero16 = jnp.zeros((16,), jnp.float32)
        one16 = jnp.ones((16,), jnp.float32)

        def fill(i, _):
            zbuf[i, 0:16] = zero16
            ones_v[i, 0:16] = one16
            return 0

        lax.fori_loop(0, _LB, fill, 0)

        for k in range(zcopies):
            pltpu.sync_copy(zbuf, deg_sh.at[pl.ds(sid * rows_per_sub + k * _LB, _LB)])
        plsc.subcore_barrier()

        full = eb % n_batches == 0
        last_rows = eb % n_batches if not full else n_batches

        @pl.when(jnp.logical_or(wid * n_batches + n_batches <= eb, full))
        def _():
            pltpu.sync_copy(ei2.at[pl.ds(wid * n_batches, n_batches)], src_v)
            pltpu.sync_copy(ei2.at[pl.ds(eb + wid * n_batches, n_batches)], dst_v)

        if not full:
            @pl.when(wid * n_batches + n_batches > eb)
            def _():
                pltpu.sync_copy(
                    ei2.at[pl.ds((_NW - 1) * n_batches, last_rows)],
                    src_v.at[pl.ds(0, last_rows)],
                )
                pltpu.sync_copy(
                    ei2.at[pl.ds(eb + (_NW - 1) * n_batches, last_rows)],
                    dst_v.at[pl.ds(0, last_rows)],
                )


        z16i = jnp.zeros((16,), jnp.int32)
        t16i = jnp.full((16,), half, jnp.int32)

        def tfill(i, _):
            as1[pl.ds(i * 16, 16)] = z16i
            bs1[pl.ds(i * 16, 16)] = z16i
            ad1[pl.ds(i * 16, 16)] = t16i
            bd1[pl.ds(i * 16, 16)] = t16i
            return 0

        lax.fori_loop(0, cap // 16, tfill, 0)

        def route_row(r, carry):
            offA, offB = carry
            pltpu.async_copy(ones_v, deg_sh.at[dst_v.at[r]], sdeg, add=True)
            for c in range(_LB // 16):
                s = src_v[r, pl.ds(c * 16, 16)]
                d = dst_v[r, pl.ds(c * 16, 16)]
                m = d < half
                plsc.store_compressed(as1.at[pl.ds(offA, 16)], s, mask=m)
                plsc.store_compressed(ad1.at[pl.ds(offA, 16)], d, mask=m)
                mb = jnp.logical_not(m)
                plsc.store_compressed(bs1.at[pl.ds(offB, 16)], s, mask=mb)
                plsc.store_compressed(bd1.at[pl.ds(offB, 16)], d - half, mask=mb)
                pop = jnp.max(plsc.all_reduce_population_count(m))
                offA = offA + pop
                offB = offB + (16 - pop)
            return offA, offB

        offA, offB = lax.fori_loop(
            0, nb_w, route_row,
            (jnp.zeros((), jnp.int32), jnp.zeros((), jnp.int32)),
        )

        def drain(j, _):
            pltpu.make_async_copy(ones_v, deg_sh.at[dst_v.at[0]], sdeg).wait()
            return 0

        lax.fori_loop(0, nb_w, drain, 0)

        def to2d(r, _):
            for c in range(_LB // 16):
                ad2[r, pl.ds(c * 16, 16)] = ad1[pl.ds(r * _LB + c * 16, 16)]
                bd2[r, pl.ds(c * 16, 16)] = bd1[pl.ds(r * _LB + c * 16, 16)]
            return 0

        lax.fori_loop(0, n_batches, to2d, 0)

        plsc.subcore_barrier()

        pltpu.sync_copy(
            deg_sh.at[pl.ds(sid * rows_per_sub, rows_per_sub)],
            degpart.at[pl.ds(cid * rows_sh + sid * rows_per_sub, rows_per_sub)],
        )
        pltpu.sync_copy(as1, lAs.at[wid])
        pltpu.sync_copy(bs1, lBs.at[wid])
        pltpu.sync_copy(ad2, lAd.at[pl.ds(wid * n_batches, n_batches)])
        pltpu.sync_copy(bd2, lBd.at[pl.ds(wid * n_batches, n_batches)])

        for i in range(_LB // 16):
            cnt_v[pl.ds(i * 16, 16)] = jnp.full((16,), offA, jnp.int32)
        pltpu.sync_copy(cnt_v, cA.at[wid])
        for i in range(_LB // 16):
            cnt_v[pl.ds(i * 16, 16)] = jnp.full((16,), offB, jnp.int32)
        pltpu.sync_copy(cnt_v, cB.at[wid])

    return pl.kernel(
        body,
        out_type=(
            jax.ShapeDtypeStruct((_NC * rows_sh, 16), jnp.float32),
            jax.ShapeDtypeStruct((_NW, cap), jnp.int32),
            jax.ShapeDtypeStruct((_NW * n_batches, _LB), jnp.int32),
            jax.ShapeDtypeStruct((_NW, cap), jnp.int32),
            jax.ShapeDtypeStruct((_NW * n_batches, _LB), jnp.int32),
            jax.ShapeDtypeStruct((_NW, _LB), jnp.int32),
            jax.ShapeDtypeStruct((_NW, _LB), jnp.int32),
        ),
        mesh=plsc.VectorSubcoreMesh(core_axis_name="c", subcore_axis_name="s"),
        scratch_types=[
            pltpu.VMEM((n_batches, _LB), jnp.int32),
            pltpu.VMEM((n_batches, _LB), jnp.int32),
            pltpu.VMEM((cap,), jnp.int32),
            pltpu.VMEM((cap,), jnp.int32),
            pltpu.VMEM((cap,), jnp.int32),
            pltpu.VMEM((cap,), jnp.int32),
            pltpu.VMEM((n_batches, _LB), jnp.int32),
            pltpu.VMEM((n_batches, _LB), jnp.int32),
            pltpu.VMEM((_LB, 16), jnp.float32),
            pltpu.VMEM((_LB, 16), jnp.float32),
            pltpu.VMEM((_LB,), jnp.int32),
            pltpu.SemaphoreType.DMA,
            pltpu.VMEM_SHARED((rows_sh, 16), jnp.float32),
        ],
        compiler_params=_UNTILED_NL,
    )


def _agg_kernel_fn(n_nodes, d, n_batches, rows_half):
    stripe = rows_half // _NS
    zrows = 160
    zcopies = stripe // zrows
    nbuf = 4
    n_per_sub = n_nodes // _NS
    half = n_nodes // 2

    def body(h128, lAs, lAd, lBs, lBd, cA, cB, ag,
             srcq_v, dstq_v, rows_v, zbuf, cnt_v, sg0, sg1, sg2, sg3,
             agg_sh, h_sh):
        cid = lax.axis_index("c")
        sid = lax.axis_index("s")
        sgs = (sg0, sg1, sg2, sg3)

        zero16 = jnp.zeros((16,), jnp.float32)

        def fill(i, _):
            for j in range(d // 16):
                zbuf[i, pl.ds(j * 16, 16)] = zero16
            return 0

        lax.fori_loop(0, zrows, fill, 0)

        pltpu.sync_copy(
            h128.at[pl.ds(sid * n_per_sub, n_per_sub), pl.ds(0, d)],
            h_sh.at[pl.ds(sid * n_per_sub, n_per_sub)],
        )
        for k in range(zcopies):
            pltpu.sync_copy(zbuf, agg_sh.at[pl.ds(sid * stripe + k * zrows, zrows)])
        plsc.subcore_barrier()

        def run_lists(Ls, Ld, Cc):
            for q in range(2):
                w = 2 * sid + q
                pltpu.sync_copy(Ls.at[w], srcq_v)
                pltpu.sync_copy(Ld.at[pl.ds(w * n_batches, n_batches)], dstq_v)
                pltpu.sync_copy(Cc.at[w], cnt_v)
                cnt = jnp.max(cnt_v[0:16])
                nbq = (cnt + _LB - 1) // _LB

                for b in range(nbuf):
                    @pl.when(b < nbq)
                    def _():
                        pltpu.async_copy(
                            h_sh.at[srcq_v.at[pl.ds(b * _LB, _LB)]],
                            rows_v.at[b], sgs[b],
                        )

                def group(jj, _):
                    for b in range(nbuf):
                        j = jj * nbuf + b

                        @pl.when(j < nbq)
                        def _():
                            pltpu.make_async_copy(
                                h_sh.at[srcq_v.at[pl.ds(j * _LB, _LB)]],
                                rows_v.at[b], sgs[b],
                            ).wait()
                            pltpu.sync_copy(
                                rows_v.at[b], agg_sh.at[dstq_v.at[j]], add=True
                            )

                        @pl.when(j + nbuf < nbq)
                        def _():
                            pltpu.async_copy(
                                h_sh.at[srcq_v.at[pl.ds((j + nbuf) * _LB, _LB)]],
                                rows_v.at[b], sgs[b],
                            )

                    return 0

                lax.fori_loop(0, n_batches // nbuf, group, 0)

        @pl.when(cid == 0)
        def _():
            run_lists(lAs, lAd, cA)

        @pl.when(cid == 1)
        def _():
            run_lists(lBs, lBd, cB)

        plsc.subcore_barrier()
        pltpu.sync_copy(
            agg_sh.at[pl.ds(sid * stripe, stripe)],
            ag.at[pl.ds(cid * rows_half + sid * stripe, stripe)],
        )

    cap = n_batches * _LB
    return pl.kernel(
        body,
        out_type=jax.ShapeDtypeStruct((_NC * rows_half, d), jnp.float32),
        mesh=plsc.VectorSubcoreMesh(core_axis_name="c", subcore_axis_name="s"),
        scratch_types=[
            pltpu.VMEM((cap,), jnp.int32),
            pltpu.VMEM((n_batches, _LB), jnp.int32),
            pltpu.VMEM((nbuf, _LB, d), jnp.float32),
            pltpu.VMEM((zrows, d), jnp.float32),
            pltpu.VMEM((_LB,), jnp.int32),
            pltpu.SemaphoreType.DMA,
            pltpu.SemaphoreType.DMA,
            pltpu.SemaphoreType.DMA,
            pltpu.SemaphoreType.DMA,
            pltpu.VMEM_SHARED((rows_half, d), jnp.float32),
            pltpu.VMEM_SHARED((n_nodes, d), jnp.float32),
        ],
        compiler_params=_UNTILED_NL,
    )


def _dis_col(dp_ref, n):
    deg = 1.0 + dp_ref[0][0:n, 0:1] + dp_ref[1][0:n, 0:1]
    return lax.rsqrt(deg)


def _tc1_body(x_ref, w1_ref, dp_ref, h1_ref):
    n, d = h1_ref.shape[0], w1_ref.shape[1]
    dis = _dis_col(dp_ref, n)
    h = jnp.dot(x_ref[...], w1_ref[...], preferred_element_type=jnp.float32)
    h1_ref[:, 0:d] = h * dis


def _agg_total(ag_ref, h_ref, n, d):
    half = n // 2
    rows_half = ag_ref.shape[0] // 2
    return (
        jnp.concatenate(
            [ag_ref[0:half], ag_ref[rows_half:rows_half + half]], axis=0
        )
        + h_ref[0:n, 0:d]
    )


def _tc2_body(ag_ref, h1_ref, dp_ref, w2_ref, b1_ref, h2_ref):
    n, d = h2_ref.shape[0], w2_ref.shape[1]
    dis = _dis_col(dp_ref, n)
    out1 = _agg_total(ag_ref, h1_ref, n, d) * dis + b1_ref[...]
    h2_ref[:, 0:d] = jnp.dot(out1, w2_ref[...], preferred_element_type=jnp.float32) * dis


def _tc3_body(ag_ref, h2_ref, dp_ref, b2_ref, out_ref):
    n, d = out_ref.shape
    dis = _dis_col(dp_ref, n)
    out_ref[...] = _agg_total(ag_ref, h2_ref, n, d) * dis + b2_ref[...]


def kernel(x, edge_index, W1, b1, W2, b2):
    n, d_in = x.shape
    d_out = W1.shape[1]
    e = edge_index.shape[1]

    n_batches = 8 * (-(-e // (_NW * _LB * 8)))
    rows_sh = _NS * _LB * (-(-(n + 1) // (_NS * _LB)))
    rows_half = _NS * 8 * (-(-(n // 2 + 1) // (_NS * 8)))
    eb = e // _LB
    assert d_out % 16 == 0 and n % 2 == 0 and n % _NS == 0
    assert e % _LB == 0 and (_NW - 1) * n_batches <= eb

    ei2 = edge_index.reshape(2 * eb, _LB)

    dp_flat, lAs, lAd, lBs, lBd, cA, cB = _deg_route_kernel_fn(
        n, n_batches, rows_sh
    )(ei2)
    dp = dp_flat.reshape(_NC, rows_sh, 16)

    agg_raw = _agg_kernel_fn(n, d_out, n_batches, rows_half)

    def agg(h128):
        return agg_raw(h128, lAs, lAd, lBs, lBd, cA, cB)

    b1r = b1.reshape(1, d_out)
    b2r = b2.reshape(1, d_out)

    h1 = pl.pallas_call(
        _tc1_body,
        out_shape=jax.ShapeDtypeStruct((n, 128), jnp.float32),
    )(x, W1, dp)

    ag1 = agg(h1)

    h2 = pl.pallas_call(
        _tc2_body,
        out_shape=jax.ShapeDtypeStruct((n, 128), jnp.float32),
    )(ag1, h1, dp, W2, b1r)

    ag2 = agg(h2)

    out = pl.pallas_call(
        _tc3_body,
        out_shape=jax.ShapeDtypeStruct((n, d_out), jnp.float32),
    )(ag2, h2, dp, b2r)

    return out
